# baseline (speedup 1.0000x reference)
import jax
import jax.numpy as jnp
from jax import lax
from jax.experimental import pallas as pl
from jax.experimental.pallas import tpu as pltpu
from jax.experimental.pallas import tpu_sc as plsc

NB = 64
OUT = 1317
B = 128
NSC = 2
TPS = 16
NTILES = NSC * TPS
EPS = 1e-16


def _mm_body(x_ref, w_ref, b_ref, o_ref):
    o_ref[...] = (
        jnp.dot(x_ref[...], w_ref[...], preferred_element_type=jnp.float32)
        + b_ref[...]
    )


def _matmul_bias(x, w, b, bm=512):
    m, k = x.shape
    _, n = w.shape
    return pl.pallas_call(
        _mm_body,
        grid=(pl.cdiv(m, bm),),
        in_specs=[
            pl.BlockSpec((bm, k), lambda i: (i, 0)),
            pl.BlockSpec((k, n), lambda i: (0, 0)),
            pl.BlockSpec((1, n), lambda i: (0, 0)),
        ],
        out_specs=pl.BlockSpec((bm, n), lambda i: (i, 0)),
        out_shape=jax.ShapeDtypeStruct((m, n), jnp.float32),
    )(x, w, b.reshape(1, n))


def _sc_attn(n, e, h, c_pad, F, nch):
    hcp = h * c_pad
    nblk = e // B
    bpt = -(-nblk // NTILES)
    grp = B // 16
    nf = F // 16
    mesh = plsc.VectorSubcoreMesh(core_axis_name="c", subcore_axis_name="s",
                                  num_cores=NSC, num_subcores=TPS)

    spc = c_pad // F

    def body(xlr_ref, ee_ref, src_ref, dst_ref, att_ref, ex_ref,
             srcb, dstb, idxs, idxd, idxe, xlb, xrb, eeb, accf, exb, attv,
             sem1, sem2, sem3):
        cid = lax.axis_index("c")
        sid = lax.axis_index("s")
        wid = sid * NSC + cid
        lanes = lax.iota(jnp.int32, 16)
        z16 = jnp.zeros((16,), jnp.float32)

        pltpu.sync_copy(att_ref, attv)

        def block_body(bi, _):
            blk = wid + NTILES * bi

            @pl.when(blk < nblk)
            def _():
                e0 = blk * B
                pltpu.sync_copy(src_ref.at[pl.ds(e0, B)], srcb)
                pltpu.sync_copy(dst_ref.at[pl.ds(e0, B)], dstb)

                def chunk(f, _):
                    def mkidx(j, _):
                        s = srcb[pl.ds(j * 16, 16)]
                        idxs[pl.ds(j * 16, 16)] = s * (2 * nch) + f
                        d = dstb[pl.ds(j * 16, 16)]
                        idxd[pl.ds(j * 16, 16)] = d * (2 * nch) + (nch + f)
                        idxe[pl.ds(j * 16, 16)] = (
                            (e0 + j * 16 + lanes) * nch + f)
                        return 0
                    lax.fori_loop(0, grp, mkidx, 0)
                    cp1 = pltpu.async_copy(xlr_ref.at[idxs], xlb, sem1)
                    cp2 = pltpu.async_copy(xlr_ref.at[idxd], xrb, sem2)
                    cp3 = pltpu.async_copy(ee_ref.at[idxe], eeb, sem3)
                    cp1.wait()
                    cp2.wait()
                    cp3.wait()
                    attw = [attv[pl.ds(f * F + j * 16, 16)]
                            for j in range(nf)]

                    def grpbody(g, _):
                        vec = z16
                        for k in range(16):
                            i = g * 16 + k
                            acc = z16
                            for j in range(nf):
                                cs = pl.ds(j * 16, 16)
                                z = xlb[i, cs] + xrb[i, cs] + eeb[i, cs]
                                z = jnp.maximum(z, z * 0.2)
                                acc = acc + z * attw[j]
                            vec = jnp.where(lanes == k, jnp.sum(acc), vec)
                        accf[pl.ds(f * B + g * 16, 16)] = vec
                        return 0
                    lax.fori_loop(0, grp, grpbody, 0)
                    return 0
                lax.fori_loop(0, nch, chunk, 0)

                def p2(g, _):
                    for hh in range(h):
                        lsum = accf[pl.ds(hh * spc * B + g * 16, 16)]
                        for sub in range(1, spc):
                            lsum = lsum + accf[
                                pl.ds((hh * spc + sub) * B + g * 16, 16)]
                        exb[hh, pl.ds(g * 16, 16)] = jnp.exp(lsum)
                    return 0
                lax.fori_loop(0, grp, p2, 0)
                pltpu.sync_copy(exb, ex_ref.at[pl.ds(0, h), pl.ds(e0, B)])
            return 0
        lax.fori_loop(0, bpt, block_body, 0)

    return pl.kernel(
        body,
        out_type=jax.ShapeDtypeStruct((h, e), jnp.float32),
        mesh=mesh,
        compiler_params=pltpu.CompilerParams(needs_layout_passes=False),
        scratch_types=[
            pltpu.VMEM((B,), jnp.int32), pltpu.VMEM((B,), jnp.int32),
            pltpu.VMEM((B,), jnp.int32), pltpu.VMEM((B,), jnp.int32),
            pltpu.VMEM((B,), jnp.int32),
            pltpu.VMEM((B, F), jnp.float32), pltpu.VMEM((B, F), jnp.float32),
            pltpu.VMEM((B, F), jnp.float32),
            pltpu.VMEM((nch * B,), jnp.float32),
            pltpu.VMEM((h, B), jnp.float32),
            pltpu.VMEM((hcp,), jnp.float32),
            pltpu.SemaphoreType.DMA, pltpu.SemaphoreType.DMA,
            pltpu.SemaphoreType.DMA,
        ],
    )


def _sc_den(n, e, h):
    nblk = e // B
    bpt = -(-nblk // NTILES)
    rows_pt = n // TPS
    grp = B // 16
    npieces = rows_pt // B
    mesh = plsc.VectorSubcoreMesh(core_axis_name="c", subcore_axis_name="s",
                                  num_cores=NSC, num_subcores=TPS)

    def body(ex_ref, dst_ref, denp_ref, dstb, exb, exT, den_sp, sem1):
        cid = lax.axis_index("c")
        sid = lax.axis_index("s")
        wid = sid * NSC + cid
        r0 = sid * rows_pt
        lanes = lax.iota(jnp.int32, 16)
        z16 = jnp.zeros((16,), jnp.float32)

        def zex(i, _):
            exT[i, :] = z16
            return 0
        lax.fori_loop(0, B, zex, 0)
        for p in range(npieces):
            pltpu.sync_copy(exT, den_sp.at[pl.ds(r0 + p * B, B)])
        plsc.subcore_barrier()

        def block_body(bi, _):
            blk = wid + NTILES * bi

            @pl.when(blk < nblk)
            def _():
                e0 = blk * B
                pltpu.sync_copy(dst_ref.at[pl.ds(e0, B)], dstb)
                pltpu.sync_copy(ex_ref.at[pl.ds(0, h), pl.ds(e0, B)], exb)

                def exrow(g, _):
                    evs = [exb[hh, pl.ds(g * 16, 16)] for hh in range(h)]
                    for k in range(16):
                        vec = z16
                        for hh in range(h):
                            vec = jnp.where(lanes == hh, evs[hh][k], vec)
                        exT[g * 16 + k, :] = vec
                    return 0
                lax.fori_loop(0, grp, exrow, 0)
                pltpu.sync_copy(exT, den_sp.at[dstb], add=True)
            return 0
        lax.fori_loop(0, bpt, block_body, 0)
        plsc.subcore_barrier()

        for p in range(npieces):
            rr = r0 + p * B
            pltpu.sync_copy(den_sp.at[pl.ds(rr, B)], exT)
            pltpu.sync_copy(exT, denp_ref.at[cid, pl.ds(rr, B)])

    return pl.kernel(
        body,
        out_type=jax.ShapeDtypeStruct((2, n, 16), jnp.float32),
        mesh=mesh,
        compiler_params=pltpu.CompilerParams(needs_layout_passes=False),
        scratch_types=[
            pltpu.VMEM((B,), jnp.int32),
            pltpu.VMEM((h, B), jnp.float32),
            pltpu.VMEM((B, 16), jnp.float32),
            pltpu.VMEM_SHARED((n, 16), jnp.float32),
            pltpu.SemaphoreType.DMA,
        ],
    )


def _sc_aggr(n, e, h, c_pad, F, nch):
    hcp = h * c_pad
    nblk = e // B
    bps = -(-nblk // TPS)
    rows_pt = n // TPS
    grp = B // 16
    Fb = 64
    nchb = 2 * nch
    nf = Fb // 16
    npass = -(-nchb // NSC)
    npieces = rows_pt // B
    mesh = plsc.VectorSubcoreMesh(core_axis_name="c", subcore_axis_name="s",
                                  num_cores=NSC, num_subcores=TPS)

    def body(xlr_ref, exsel_ref, densel_ref, src_ref, dst_ref, bias_ref,
             y_ref, srcb, dstb, idxs, rowsb, halfb, exb1, biasv, denb1,
             out_sp, sem1):
        cid = lax.axis_index("c")
        sid = lax.axis_index("s")
        r0 = sid * rows_pt
        z16 = jnp.zeros((16,), jnp.float32)

        pltpu.sync_copy(bias_ref, biasv)

        def zrow(i, _):
            for j in range(nf):
                halfb[i, pl.ds(j * 16, 16)] = z16
            return 0

        def fpass(fp, _):
            f = cid + NSC * fp
            off = (f % 2) * Fb

            lax.fori_loop(0, B, zrow, 0)
            for p in range(npieces):
                pltpu.sync_copy(halfb, out_sp.at[pl.ds(r0 + p * B, B)])
            plsc.subcore_barrier()

            def block_body(bi, _):
                blk = sid + TPS * bi

                @pl.when(blk < nblk)
                def _():
                    e0 = blk * B
                    pltpu.sync_copy(src_ref.at[pl.ds(e0, B)], srcb)
                    pltpu.sync_copy(dst_ref.at[pl.ds(e0, B)], dstb)

                    def mkidx(j, _):
                        s = srcb[pl.ds(j * 16, 16)]
                        idxs[pl.ds(j * 16, 16)] = s * (2 * nch) + (f // 2)
                        return 0
                    lax.fori_loop(0, grp, mkidx, 0)
                    pltpu.async_copy(xlr_ref.at[idxs], rowsb, sem1).wait()
                    pltpu.sync_copy(
                        exsel_ref.at[f, 0, pl.ds(e0, B)], exb1)

                    def scale(g, _):
                        ev = exb1[pl.ds(g * 16, 16)]
                        for k in range(16):
                            a = ev[k]
                            i = g * 16 + k
                            for j in range(nf):
                                v = rowsb[i, pl.ds(off + j * 16, 16)]
                                halfb[i, pl.ds(j * 16, 16)] = v * a
                        return 0
                    lax.fori_loop(0, grp, scale, 0)
                    pltpu.sync_copy(halfb, out_sp.at[dstb], add=True)
                return 0
            lax.fori_loop(0, bps, block_body, 0)
            plsc.subcore_barrier()

            for p in range(npieces):
                rr = r0 + p * B
                pltpu.sync_copy(out_sp.at[pl.ds(rr, B)], halfb)
                pltpu.sync_copy(densel_ref.at[f, 0, pl.ds(rr, B)], denb1)
                bvs = [biasv[pl.ds(f * Fb + j * 16, 16)]
                       for j in range(nf)]

                def bgrp(g, _):
                    dvv = denb1[pl.ds(g * 16, 16)]
                    for k in range(16):
                        dv = dvv[k] + EPS
                        i = g * 16 + k
                        for j in range(nf):
                            v = halfb[i, pl.ds(j * 16, 16)] / dv + bvs[j]
                            v = jnp.where(v > 0, v, jnp.exp(v) - 1.0)
                            halfb[i, pl.ds(j * 16, 16)] = v
                    return 0
                lax.fori_loop(0, grp, bgrp, 0)
                pltpu.sync_copy(halfb, y_ref.at[f, pl.ds(rr, B)])
            plsc.subcore_barrier()
            return 0
        lax.fori_loop(0, npass, fpass, 0)

    return pl.kernel(
        body,
        out_type=jax.ShapeDtypeStruct((nchb, n, Fb), jnp.float32),
        mesh=mesh,
        compiler_params=pltpu.CompilerParams(needs_layout_passes=False),
        scratch_types=[
            pltpu.VMEM((B,), jnp.int32), pltpu.VMEM((B,), jnp.int32),
            pltpu.VMEM((B,), jnp.int32),
            pltpu.VMEM((B, F), jnp.float32),
            pltpu.VMEM((B, Fb), jnp.float32),
            pltpu.VMEM((B,), jnp.float32),
            pltpu.VMEM((hcp,), jnp.float32),
            pltpu.VMEM((B,), jnp.float32),
            pltpu.VMEM_SHARED((n, Fb), jnp.float32),
            pltpu.SemaphoreType.DMA,
        ],
    )


BN = 512


def _pool_body(ids_ref, x_ref, s_ref, c_ref):
    i = pl.program_id(0)
    ids = ids_ref[0, 0, :]
    oh = (lax.broadcasted_iota(jnp.int32, (NB, BN), 0)
          == ids[None, :]).astype(jnp.float32)
    ps = jnp.dot(oh, x_ref[...], preferred_element_type=jnp.float32)
    pc = jnp.sum(oh, axis=1)

    @pl.when(i == 0)
    def _():
        s_ref[...] = jnp.zeros_like(s_ref)
        c_ref[...] = jnp.zeros_like(c_ref)
    s_ref[...] += ps
    c_ref[...] += jnp.broadcast_to(pc[:, None], c_ref.shape)


def _pool_sums(x, batch):
    n, d = x.shape
    ids3 = batch.reshape(n // BN, 1, BN)
    return pl.pallas_call(
        _pool_body,
        grid=(n // BN,),
        in_specs=[
            pl.BlockSpec((1, 1, BN), lambda i: (i, 0, 0)),
            pl.BlockSpec((BN, d), lambda i: (i, 0)),
        ],
        out_specs=[
            pl.BlockSpec((NB, d), lambda i: (0, 0)),
            pl.BlockSpec((NB, 128), lambda i: (0, 0)),
        ],
        out_shape=[jax.ShapeDtypeStruct((NB, d), jnp.float32),
                   jax.ShapeDtypeStruct((NB, 128), jnp.float32)],
    )(ids3, x)


def _comb_body(ss_ref, sc_ref, ts_ref, tc_ref, o1_ref, o2_ref):
    x = (ss_ref[...] / jnp.maximum(sc_ref[:, 0:1], 1.0)
         + ts_ref[...] / jnp.maximum(tc_ref[:, 0:1], 1.0))
    o1_ref[...] = x
    o2_ref[...] = jax.nn.sigmoid(x)


def _combine(ss, sc, ts, tc):
    d = ss.shape[1]
    return pl.pallas_call(
        _comb_body,
        out_shape=[jax.ShapeDtypeStruct((NB, d), jnp.float32),
                   jax.ShapeDtypeStruct((NB, d), jnp.float32)],
    )(ss, sc, ts, tc)


def _pad_cols(w, tgt):
    return jnp.pad(w, ((0, 0), (0, tgt - w.shape[1])))


def _gat_layer(x, src, dst, ea, Wl, bl, Wr, br, We, att, bias):
    n = x.shape[0]
    e = src.shape[0]
    h, c = att.shape
    if h == 4:
        c_pad, F, nch = 256, 128, 8
    else:
        c_pad, F, nch = 1408, 128, 11
    hcp = h * c_pad

    Wlp = _pad_cols(Wl, hcp)
    Wrp = _pad_cols(Wr, hcp)
    W2 = jnp.concatenate([Wlp, Wrp], axis=1)
    b2 = jnp.concatenate([
        jnp.pad(bl, (0, hcp - bl.shape[0])),
        jnp.pad(br, (0, hcp - br.shape[0])),
    ])
    xlr = _matmul_bias(x, W2, b2)
    xlr_r = xlr.reshape(n * 2 * nch, F)

    ea_p = jnp.pad(ea, ((0, 0), (0, 16 - ea.shape[1])))
    We_p = _pad_cols(jnp.pad(We, ((0, 16 - We.shape[0]), (0, 0))), hcp)
    ee = _matmul_bias(ea_p, We_p, jnp.zeros((hcp,), jnp.float32), bm=2048)
    ee_r = ee.reshape(e * nch, F)

    att_flat = jnp.pad(att.reshape(-1), (0, hcp - h * c))
    bias_p = jnp.pad(bias, (0, hcp - bias.shape[0]))

    ex = _sc_attn(n, e, h, c_pad, F, nch)(xlr_r, ee_r, src, dst, att_flat)
    denP = _sc_den(n, e, h)(ex, dst)
    nchb = 2 * nch
    head_map = jnp.array([(f * 64) // c_pad for f in range(nchb)],
                         dtype=jnp.int32)
    exsel = ex[head_map].reshape(nchb, 1, e)
    den = (denP[0] + denP[1])[:, :h]
    densel = den.T[head_map].reshape(nchb, 1, n)
    y3 = _sc_aggr(n, e, h, c_pad, F, nch)(
        xlr_r, exsel, densel, src, dst, bias_p)
    return y3.transpose(1, 0, 2).reshape(n, hcp)


def _branch(x, ei, ea, params):
    src, dst = ei[0], ei[1]
    for (Wl, bl, Wr, br, We, att, bias) in params:
        x = _gat_layer(x, src, dst, ea, Wl, bl, Wr, br, We, att, bias)
    return x


def kernel(x_s, edge_index_s, edge_attr_s, x_t, edge_index_t, edge_attr_t, xs_batch, xt_batch, s1_Wl, s1_bl, s1_Wr, s1_br, s1_We, s1_att, s1_bias, s2_Wl, s2_bl, s2_Wr, s2_br, s2_We, s2_att, s2_bias, s3_Wl, s3_bl, s3_Wr, s3_br, s3_We, s3_att, s3_bias, t1_Wl, t1_bl, t1_Wr, t1_br, t1_We, t1_att, t1_bias, t2_Wl, t2_bl, t2_Wr, t2_br, t2_We, t2_att, t2_bias, t3_Wl, t3_bl, t3_Wr, t3_br, t3_We, t3_att, t3_bias):
    ps = [
        (s1_Wl, s1_bl, s1_Wr, s1_br, s1_We, s1_att, s1_bias),
        (s2_Wl, s2_bl, s2_Wr, s2_br, s2_We, s2_att, s2_bias),
        (s3_Wl, s3_bl, s3_Wr, s3_br, s3_We, s3_att, s3_bias),
    ]
    pt = [
        (t1_Wl, t1_bl, t1_Wr, t1_br, t1_We, t1_att, t1_bias),
        (t2_Wl, t2_bl, t2_Wr, t2_br, t2_We, t2_att, t2_bias),
        (t3_Wl, t3_bl, t3_Wr, t3_br, t3_We, t3_att, t3_bias),
    ]
    npad = 240
    x_s = jnp.pad(x_s, ((0, npad), (0, 0)))
    x_t = jnp.pad(x_t, ((0, npad), (0, 0)))
    xs_batch = jnp.pad(xs_batch, (0, npad), constant_values=NB)
    xt_batch = jnp.pad(xt_batch, (0, npad), constant_values=NB)
    xs = _branch(x_s, edge_index_s, edge_attr_s, ps)
    xt = _branch(x_t, edge_index_t, edge_attr_t, pt)
    ss, sc = _pool_sums(xs, xs_batch)
    ts, tc = _pool_sums(xt, xt_batch)
    x, sg = _combine(ss, sc, ts, tc)
    return (x[:, :OUT], sg[:, :OUT])


# speedup vs baseline: 1.0668x; 1.0668x over previous
import jax
import jax.numpy as jnp
from jax import lax
from jax.experimental import pallas as pl
from jax.experimental.pallas import tpu as pltpu
from jax.experimental.pallas import tpu_sc as plsc

NB = 64
OUT = 1317
B = 128
NSC = 2
TPS = 16
NTILES = NSC * TPS
EPS = 1e-16


def _mm_body(x_ref, w_ref, b_ref, o_ref):
    o_ref[...] = (
        jnp.dot(x_ref[...], w_ref[...], preferred_element_type=jnp.float32)
        + b_ref[...]
    )


def _matmul_bias(x, w, b, bm=512):
    m, k = x.shape
    _, n = w.shape
    return pl.pallas_call(
        _mm_body,
        grid=(pl.cdiv(m, bm),),
        in_specs=[
            pl.BlockSpec((bm, k), lambda i: (i, 0)),
            pl.BlockSpec((k, n), lambda i: (0, 0)),
            pl.BlockSpec((1, n), lambda i: (0, 0)),
        ],
        out_specs=pl.BlockSpec((bm, n), lambda i: (i, 0)),
        out_shape=jax.ShapeDtypeStruct((m, n), jnp.float32),
    )(x, w, b.reshape(1, n))


def _sc_attn(n, e, h, c_pad, F, nch):
    hcp = h * c_pad
    nblk = e // B
    bpt = -(-nblk // NTILES)
    grp = B // 16
    nf = F // 16
    mesh = plsc.VectorSubcoreMesh(core_axis_name="c", subcore_axis_name="s",
                                  num_cores=NSC, num_subcores=TPS)

    def body(xlr_ref, ee_ref, src_ref, dst_ref, att_ref, ex_ref,
             srcb, dstb, idxs, idxd, idxe, xlb, xrb, eeb, logb, exb, attv,
             sem1, sem2, sem3):
        cid = lax.axis_index("c")
        sid = lax.axis_index("s")
        wid = sid * NSC + cid
        lanes = lax.iota(jnp.int32, 16)
        z16 = jnp.zeros((16,), jnp.float32)

        pltpu.sync_copy(att_ref, attv)

        def block_body(bi, _):
            blk = wid + NTILES * bi

            @pl.when(blk < nblk)
            def _():
                e0 = blk * B
                pltpu.sync_copy(src_ref.at[pl.ds(e0, B)], srcb)
                pltpu.sync_copy(dst_ref.at[pl.ds(e0, B)], dstb)
                for hh in range(h):
                    def zrow(g, _):
                        logb[hh, pl.ds(g * 16, 16)] = z16
                        return 0
                    lax.fori_loop(0, grp, zrow, 0)

                for f in range(nch):
                    h0 = (f * F) // c_pad

                    def mkidx(j, _):
                        s = srcb[pl.ds(j * 16, 16)]
                        idxs[pl.ds(j * 16, 16)] = s * (2 * nch) + f
                        d = dstb[pl.ds(j * 16, 16)]
                        idxd[pl.ds(j * 16, 16)] = d * (2 * nch) + (nch + f)
                        idxe[pl.ds(j * 16, 16)] = (
                            (e0 + j * 16 + lanes) * nch + f)
                        return 0
                    lax.fori_loop(0, grp, mkidx, 0)
                    cp1 = pltpu.async_copy(xlr_ref.at[idxs], xlb, sem1)
                    cp2 = pltpu.async_copy(xlr_ref.at[idxd], xrb, sem2)
                    cp3 = pltpu.async_copy(ee_ref.at[idxe], eeb, sem3)
                    cp1.wait()
                    cp2.wait()
                    cp3.wait()

                    def grpbody(g, _):
                        def edge4(k, vec):
                            for kk in range(4):
                                i = g * 16 + k * 4 + kk
                                acc = z16
                                for j in range(nf):
                                    cs = pl.ds(j * 16, 16)
                                    z = xlb[i, cs] + xrb[i, cs] + eeb[i, cs]
                                    z = jnp.maximum(z, z * 0.2)
                                    acc = acc + z * attv[
                                        pl.ds(f * F + j * 16, 16)]
                                vec = jnp.where(
                                    lanes == k * 4 + kk, jnp.sum(acc), vec)
                            return vec
                        vec = lax.fori_loop(0, 4, edge4, z16)
                        cur = logb[h0, pl.ds(g * 16, 16)]
                        logb[h0, pl.ds(g * 16, 16)] = cur + vec
                        return 0
                    lax.fori_loop(0, grp, grpbody, 0)

                def p2(g, _):
                    for hh in range(h):
                        ev = jnp.exp(logb[hh, pl.ds(g * 16, 16)])
                        exb[hh, pl.ds(g * 16, 16)] = ev
                    return 0
                lax.fori_loop(0, grp, p2, 0)
                pltpu.sync_copy(exb, ex_ref.at[pl.ds(0, h), pl.ds(e0, B)])
            return 0
        lax.fori_loop(0, bpt, block_body, 0)

    return pl.kernel(
        body,
        out_type=jax.ShapeDtypeStruct((h, e), jnp.float32),
        mesh=mesh,
        compiler_params=pltpu.CompilerParams(needs_layout_passes=False),
        scratch_types=[
            pltpu.VMEM((B,), jnp.int32), pltpu.VMEM((B,), jnp.int32),
            pltpu.VMEM((B,), jnp.int32), pltpu.VMEM((B,), jnp.int32),
            pltpu.VMEM((B,), jnp.int32),
            pltpu.VMEM((B, F), jnp.float32), pltpu.VMEM((B, F), jnp.float32),
            pltpu.VMEM((B, F), jnp.float32),
            pltpu.VMEM((h, B), jnp.float32), pltpu.VMEM((h, B), jnp.float32),
            pltpu.VMEM((hcp,), jnp.float32),
            pltpu.SemaphoreType.DMA, pltpu.SemaphoreType.DMA,
            pltpu.SemaphoreType.DMA,
        ],
    )


def _sc_den(n, e, h):
    nblk = e // B
    bpt = -(-nblk // NTILES)
    rows_pt = n // TPS
    grp = B // 16
    npieces = rows_pt // B
    mesh = plsc.VectorSubcoreMesh(core_axis_name="c", subcore_axis_name="s",
                                  num_cores=NSC, num_subcores=TPS)

    def body(ex_ref, dst_ref, denp_ref, dstb, exb, exT, den_sp, sem1):
        cid = lax.axis_index("c")
        sid = lax.axis_index("s")
        wid = sid * NSC + cid
        r0 = sid * rows_pt
        lanes = lax.iota(jnp.int32, 16)
        z16 = jnp.zeros((16,), jnp.float32)

        def zex(i, _):
            exT[i, :] = z16
            return 0
        lax.fori_loop(0, B, zex, 0)
        for p in range(npieces):
            pltpu.sync_copy(exT, den_sp.at[pl.ds(r0 + p * B, B)])
        plsc.subcore_barrier()

        def block_body(bi, _):
            blk = wid + NTILES * bi

            @pl.when(blk < nblk)
            def _():
                e0 = blk * B
                pltpu.sync_copy(dst_ref.at[pl.ds(e0, B)], dstb)
                pltpu.sync_copy(ex_ref.at[pl.ds(0, h), pl.ds(e0, B)], exb)

                def exrow(g, _):
                    evs = [exb[hh, pl.ds(g * 16, 16)] for hh in range(h)]
                    for k in range(16):
                        vec = z16
                        for hh in range(h):
                            vec = jnp.where(lanes == hh, evs[hh][k], vec)
                        exT[g * 16 + k, :] = vec
                    return 0
                lax.fori_loop(0, grp, exrow, 0)
                pltpu.sync_copy(exT, den_sp.at[dstb], add=True)
            return 0
        lax.fori_loop(0, bpt, block_body, 0)
        plsc.subcore_barrier()

        for p in range(npieces):
            rr = r0 + p * B
            pltpu.sync_copy(den_sp.at[pl.ds(rr, B)], exT)
            pltpu.sync_copy(exT, denp_ref.at[cid, pl.ds(rr, B)])

    return pl.kernel(
        body,
        out_type=jax.ShapeDtypeStruct((2, n, 16), jnp.float32),
        mesh=mesh,
        compiler_params=pltpu.CompilerParams(needs_layout_passes=False),
        scratch_types=[
            pltpu.VMEM((B,), jnp.int32),
            pltpu.VMEM((h, B), jnp.float32),
            pltpu.VMEM((B, 16), jnp.float32),
            pltpu.VMEM_SHARED((n, 16), jnp.float32),
            pltpu.SemaphoreType.DMA,
        ],
    )


def _sc_aggr(n, e, h, c_pad, F, nch):
    hcp = h * c_pad
    nblk = e // B
    bps = -(-nblk // TPS)
    rows_pt = n // TPS
    grp = B // 16
    Fb = 64
    nchb = 2 * nch
    nf = Fb // 16
    npass = -(-nchb // NSC)
    npieces = rows_pt // B
    mesh = plsc.VectorSubcoreMesh(core_axis_name="c", subcore_axis_name="s",
                                  num_cores=NSC, num_subcores=TPS)

    def body(xlr_ref, exsel_ref, densel_ref, src_ref, dst_ref, bias_ref,
             y_ref, srcb, dstb, idxs, rowsb, halfb, exb1, biasv, denb1,
             out_sp, sem1):
        cid = lax.axis_index("c")
        sid = lax.axis_index("s")
        r0 = sid * rows_pt
        z16 = jnp.zeros((16,), jnp.float32)

        pltpu.sync_copy(bias_ref, biasv)

        def zrow(i, _):
            for j in range(nf):
                halfb[i, pl.ds(j * 16, 16)] = z16
            return 0

        def fpass(fp, _):
            f = cid + NSC * fp
            off = (f % 2) * Fb

            lax.fori_loop(0, B, zrow, 0)
            for p in range(npieces):
                pltpu.sync_copy(halfb, out_sp.at[pl.ds(r0 + p * B, B)])
            plsc.subcore_barrier()

            def block_body(bi, _):
                blk = sid + TPS * bi

                @pl.when(blk < nblk)
                def _():
                    e0 = blk * B
                    pltpu.sync_copy(src_ref.at[pl.ds(e0, B)], srcb)
                    pltpu.sync_copy(dst_ref.at[pl.ds(e0, B)], dstb)

                    def mkidx(j, _):
                        s = srcb[pl.ds(j * 16, 16)]
                        idxs[pl.ds(j * 16, 16)] = s * (2 * nch) + (f // 2)
                        return 0
                    lax.fori_loop(0, grp, mkidx, 0)
                    pltpu.async_copy(xlr_ref.at[idxs], rowsb, sem1).wait()
                    pltpu.sync_copy(
                        exsel_ref.at[f, 0, pl.ds(e0, B)], exb1)

                    def scale(g, _):
                        ev = exb1[pl.ds(g * 16, 16)]
                        for k in range(16):
                            a = ev[k]
                            i = g * 16 + k
                            for j in range(nf):
                                v = rowsb[i, pl.ds(off + j * 16, 16)]
                                halfb[i, pl.ds(j * 16, 16)] = v * a
                        return 0
                    lax.fori_loop(0, grp, scale, 0)
                    pltpu.sync_copy(halfb, out_sp.at[dstb], add=True)
                return 0
            lax.fori_loop(0, bps, block_body, 0)
            plsc.subcore_barrier()

            for p in range(npieces):
                rr = r0 + p * B
                pltpu.sync_copy(out_sp.at[pl.ds(rr, B)], halfb)
                pltpu.sync_copy(densel_ref.at[f, 0, pl.ds(rr, B)], denb1)
                bvs = [biasv[pl.ds(f * Fb + j * 16, 16)]
                       for j in range(nf)]

                def bgrp(g, _):
                    dvv = denb1[pl.ds(g * 16, 16)]
                    for k in range(16):
                        dv = dvv[k] + EPS
                        i = g * 16 + k
                        for j in range(nf):
                            v = halfb[i, pl.ds(j * 16, 16)] / dv + bvs[j]
                            v = jnp.where(v > 0, v, jnp.exp(v) - 1.0)
                            halfb[i, pl.ds(j * 16, 16)] = v
                    return 0
                lax.fori_loop(0, grp, bgrp, 0)
                pltpu.sync_copy(halfb, y_ref.at[f, pl.ds(rr, B)])
            plsc.subcore_barrier()
            return 0
        lax.fori_loop(0, npass, fpass, 0)

    return pl.kernel(
        body,
        out_type=jax.ShapeDtypeStruct((nchb, n, Fb), jnp.float32),
        mesh=mesh,
        compiler_params=pltpu.CompilerParams(needs_layout_passes=False),
        scratch_types=[
            pltpu.VMEM((B,), jnp.int32), pltpu.VMEM((B,), jnp.int32),
            pltpu.VMEM((B,), jnp.int32),
            pltpu.VMEM((B, F), jnp.float32),
            pltpu.VMEM((B, Fb), jnp.float32),
            pltpu.VMEM((B,), jnp.float32),
            pltpu.VMEM((hcp,), jnp.float32),
            pltpu.VMEM((B,), jnp.float32),
            pltpu.VMEM_SHARED((n, Fb), jnp.float32),
            pltpu.SemaphoreType.DMA,
        ],
    )


BN = 512


def _pool_body(ids_ref, x_ref, s_ref, c_ref):
    i = pl.program_id(0)
    ids = ids_ref[0, 0, :]
    oh = (lax.broadcasted_iota(jnp.int32, (NB, BN), 0)
          == ids[None, :]).astype(jnp.float32)
    ps = jnp.dot(oh, x_ref[...], preferred_element_type=jnp.float32)
    pc = jnp.sum(oh, axis=1)

    @pl.when(i == 0)
    def _():
        s_ref[...] = jnp.zeros_like(s_ref)
        c_ref[...] = jnp.zeros_like(c_ref)
    s_ref[...] += ps
    c_ref[...] += jnp.broadcast_to(pc[:, None], c_ref.shape)


def _pool_sums(x, batch):
    n, d = x.shape
    ids3 = batch.reshape(n // BN, 1, BN)
    return pl.pallas_call(
        _pool_body,
        grid=(n // BN,),
        in_specs=[
            pl.BlockSpec((1, 1, BN), lambda i: (i, 0, 0)),
            pl.BlockSpec((BN, d), lambda i: (i, 0)),
        ],
        out_specs=[
            pl.BlockSpec((NB, d), lambda i: (0, 0)),
            pl.BlockSpec((NB, 128), lambda i: (0, 0)),
        ],
        out_shape=[jax.ShapeDtypeStruct((NB, d), jnp.float32),
                   jax.ShapeDtypeStruct((NB, 128), jnp.float32)],
    )(ids3, x)


def _comb_body(ss_ref, sc_ref, ts_ref, tc_ref, o1_ref, o2_ref):
    x = (ss_ref[...] / jnp.maximum(sc_ref[:, 0:1], 1.0)
         + ts_ref[...] / jnp.maximum(tc_ref[:, 0:1], 1.0))
    o1_ref[...] = x
    o2_ref[...] = jax.nn.sigmoid(x)


def _combine(ss, sc, ts, tc):
    d = ss.shape[1]
    return pl.pallas_call(
        _comb_body,
        out_shape=[jax.ShapeDtypeStruct((NB, d), jnp.float32),
                   jax.ShapeDtypeStruct((NB, d), jnp.float32)],
    )(ss, sc, ts, tc)


def _pad_cols(w, tgt):
    return jnp.pad(w, ((0, 0), (0, tgt - w.shape[1])))


def _gat_layer(x, src, dst, ea, Wl, bl, Wr, br, We, att, bias):
    n = x.shape[0]
    e = src.shape[0]
    h, c = att.shape
    if h == 4:
        c_pad, F, nch = 256, 128, 8
    else:
        c_pad, F, nch = 1408, 128, 11
    hcp = h * c_pad

    Wlp = _pad_cols(Wl, hcp)
    Wrp = _pad_cols(Wr, hcp)
    W2 = jnp.concatenate([Wlp, Wrp], axis=1)
    b2 = jnp.concatenate([
        jnp.pad(bl, (0, hcp - bl.shape[0])),
        jnp.pad(br, (0, hcp - br.shape[0])),
    ])
    xlr = _matmul_bias(x, W2, b2)
    xlr_r = xlr.reshape(n * 2 * nch, F)

    ea_p = jnp.pad(ea, ((0, 0), (0, 16 - ea.shape[1])))
    We_p = _pad_cols(jnp.pad(We, ((0, 16 - We.shape[0]), (0, 0))), hcp)
    ee = _matmul_bias(ea_p, We_p, jnp.zeros((hcp,), jnp.float32), bm=2048)
    ee_r = ee.reshape(e * nch, F)

    att_flat = jnp.pad(att.reshape(-1), (0, hcp - h * c))
    bias_p = jnp.pad(bias, (0, hcp - bias.shape[0]))

    ex = _sc_attn(n, e, h, c_pad, F, nch)(xlr_r, ee_r, src, dst, att_flat)
    denP = _sc_den(n, e, h)(ex, dst)
    nchb = 2 * nch
    head_map = jnp.array([(f * 64) // c_pad for f in range(nchb)],
                         dtype=jnp.int32)
    exsel = ex[head_map].reshape(nchb, 1, e)
    den = (denP[0] + denP[1])[:, :h]
    densel = den.T[head_map].reshape(nchb, 1, n)
    y3 = _sc_aggr(n, e, h, c_pad, F, nch)(
        xlr_r, exsel, densel, src, dst, bias_p)
    return y3.transpose(1, 0, 2).reshape(n, hcp)


def _branch(x, ei, ea, params):
    src, dst = ei[0], ei[1]
    for (Wl, bl, Wr, br, We, att, bias) in params:
        x = _gat_layer(x, src, dst, ea, Wl, bl, Wr, br, We, att, bias)
    return x


def kernel(x_s, edge_index_s, edge_attr_s, x_t, edge_index_t, edge_attr_t, xs_batch, xt_batch, s1_Wl, s1_bl, s1_Wr, s1_br, s1_We, s1_att, s1_bias, s2_Wl, s2_bl, s2_Wr, s2_br, s2_We, s2_att, s2_bias, s3_Wl, s3_bl, s3_Wr, s3_br, s3_We, s3_att, s3_bias, t1_Wl, t1_bl, t1_Wr, t1_br, t1_We, t1_att, t1_bias, t2_Wl, t2_bl, t2_Wr, t2_br, t2_We, t2_att, t2_bias, t3_Wl, t3_bl, t3_Wr, t3_br, t3_We, t3_att, t3_bias):
    ps = [
        (s1_Wl, s1_bl, s1_Wr, s1_br, s1_We, s1_att, s1_bias),
        (s2_Wl, s2_bl, s2_Wr, s2_br, s2_We, s2_att, s2_bias),
        (s3_Wl, s3_bl, s3_Wr, s3_br, s3_We, s3_att, s3_bias),
    ]
    pt = [
        (t1_Wl, t1_bl, t1_Wr, t1_br, t1_We, t1_att, t1_bias),
        (t2_Wl, t2_bl, t2_Wr, t2_br, t2_We, t2_att, t2_bias),
        (t3_Wl, t3_bl, t3_Wr, t3_br, t3_We, t3_att, t3_bias),
    ]
    npad = 240
    x_s = jnp.pad(x_s, ((0, npad), (0, 0)))
    x_t = jnp.pad(x_t, ((0, npad), (0, 0)))
    xs_batch = jnp.pad(xs_batch, (0, npad), constant_values=NB)
    xt_batch = jnp.pad(xt_batch, (0, npad), constant_values=NB)
    xs = _branch(x_s, edge_index_s, edge_attr_s, ps)
    xt = _branch(x_t, edge_index_t, edge_attr_t, pt)
    ss, sc = _pool_sums(xs, xs_batch)
    ts, tc = _pool_sums(xt, xt_batch)
    x, sg = _combine(ss, sc, ts, tc)
    return (x[:, :OUT], sg[:, :OUT])


# speedup vs baseline: 1.1647x; 1.0918x over previous
import jax
import jax.numpy as jnp
from jax import lax
from jax.experimental import pallas as pl
from jax.experimental.pallas import tpu as pltpu
from jax.experimental.pallas import tpu_sc as plsc

NB = 64
OUT = 1317
B = 128
NSC = 2
TPS = 16
NTILES = NSC * TPS
EPS = 1e-16


def _mm_body(x_ref, w_ref, b_ref, o_ref):
    o_ref[...] = (
        jnp.dot(x_ref[...], w_ref[...], preferred_element_type=jnp.float32)
        + b_ref[...]
    )


def _matmul_bias(x, w, b, bm=512):
    m, k = x.shape
    _, n = w.shape
    return pl.pallas_call(
        _mm_body,
        grid=(pl.cdiv(m, bm),),
        in_specs=[
            pl.BlockSpec((bm, k), lambda i: (i, 0)),
            pl.BlockSpec((k, n), lambda i: (0, 0)),
            pl.BlockSpec((1, n), lambda i: (0, 0)),
        ],
        out_specs=pl.BlockSpec((bm, n), lambda i: (i, 0)),
        out_shape=jax.ShapeDtypeStruct((m, n), jnp.float32),
    )(x, w, b.reshape(1, n))


def _sc_attn(n, e, h, c_pad, F, nch):
    hcp = h * c_pad
    nblk = e // B
    bpt = -(-nblk // NTILES)
    grp = B // 16
    nf = F // 16
    mesh = plsc.VectorSubcoreMesh(core_axis_name="c", subcore_axis_name="s",
                                  num_cores=NSC, num_subcores=TPS)

    def body(xlr_ref, ee_ref, src_ref, dst_ref, att_ref, ex_ref,
             srcb, dstb, idxs0, idxd0, idxe0, idxs1, idxd1, idxe1,
             xlb0, xrb0, eeb0, xlb1, xrb1, eeb1, logb, exb, attv,
             s10, s20, s30, s11, s21, s31):
        cid = lax.axis_index("c")
        sid = lax.axis_index("s")
        wid = sid * NSC + cid
        lanes = lax.iota(jnp.int32, 16)
        z16 = jnp.zeros((16,), jnp.float32)

        idxbufs = [(idxs0, idxd0, idxe0), (idxs1, idxd1, idxe1)]
        rowbufs = [(xlb0, xrb0, eeb0), (xlb1, xrb1, eeb1)]
        sems = [(s10, s20, s30), (s11, s21, s31)]

        pltpu.sync_copy(att_ref, attv)

        def block_body(bi, _):
            blk = wid + NTILES * bi

            @pl.when(blk < nblk)
            def _():
                e0 = blk * B
                pltpu.sync_copy(src_ref.at[pl.ds(e0, B)], srcb)
                pltpu.sync_copy(dst_ref.at[pl.ds(e0, B)], dstb)
                for hh in range(h):
                    def zrow(g, _):
                        logb[hh, pl.ds(g * 16, 16)] = z16
                        return 0
                    lax.fori_loop(0, grp, zrow, 0)

                def issue(f):
                    b = f % 2
                    idxs, idxd, idxe = idxbufs[b]
                    xlb, xrb, eeb = rowbufs[b]
                    sa, sb, sc = sems[b]

                    def mkidx(j, _):
                        s = srcb[pl.ds(j * 16, 16)]
                        idxs[pl.ds(j * 16, 16)] = s * (2 * nch) + f
                        d = dstb[pl.ds(j * 16, 16)]
                        idxd[pl.ds(j * 16, 16)] = d * (2 * nch) + (nch + f)
                        idxe[pl.ds(j * 16, 16)] = (
                            (e0 + j * 16 + lanes) * nch + f)
                        return 0
                    lax.fori_loop(0, grp, mkidx, 0)
                    return (pltpu.async_copy(xlr_ref.at[idxs], xlb, sa),
                            pltpu.async_copy(xlr_ref.at[idxd], xrb, sb),
                            pltpu.async_copy(ee_ref.at[idxe], eeb, sc))

                cps = issue(0)
                for f in range(nch):
                    h0 = (f * F) // c_pad
                    xlb, xrb, eeb = rowbufs[f % 2]
                    for cp in cps:
                        cp.wait()
                    if f + 1 < nch:
                        cps = issue(f + 1)

                    def grpbody(g, _):
                        def edge(k, vec):
                            i = g * 16 + k
                            acc = z16
                            for j in range(nf):
                                cs = pl.ds(j * 16, 16)
                                z = xlb[i, cs] + xrb[i, cs] + eeb[i, cs]
                                z = jnp.maximum(z, z * 0.2)
                                acc = acc + z * attv[
                                    pl.ds(f * F + j * 16, 16)]
                            s = jnp.sum(acc)
                            return jnp.where(lanes == k, s, vec)
                        vec = lax.fori_loop(0, 16, edge, z16)
                        cur = logb[h0, pl.ds(g * 16, 16)]
                        logb[h0, pl.ds(g * 16, 16)] = cur + vec
                        return 0
                    lax.fori_loop(0, grp, grpbody, 0)

                def p2(g, _):
                    for hh in range(h):
                        ev = jnp.exp(logb[hh, pl.ds(g * 16, 16)])
                        exb[hh, pl.ds(g * 16, 16)] = ev
                    return 0
                lax.fori_loop(0, grp, p2, 0)
                pltpu.sync_copy(exb, ex_ref.at[pl.ds(0, h), pl.ds(e0, B)])
            return 0
        lax.fori_loop(0, bpt, block_body, 0)

    return pl.kernel(
        body,
        out_type=jax.ShapeDtypeStruct((h, e), jnp.float32),
        mesh=mesh,
        compiler_params=pltpu.CompilerParams(needs_layout_passes=False),
        scratch_types=(
            [pltpu.VMEM((B,), jnp.int32)] * 2
            + [pltpu.VMEM((B,), jnp.int32)] * 6
            + [pltpu.VMEM((B, F), jnp.float32)] * 6
            + [pltpu.VMEM((h, B), jnp.float32)] * 2
            + [pltpu.VMEM((hcp,), jnp.float32)]
            + [pltpu.SemaphoreType.DMA] * 6
        ),
    )


def _sc_den(n, e, h):
    nblk = e // B
    bpt = -(-nblk // NTILES)
    rows_pt = n // TPS
    grp = B // 16
    npieces = rows_pt // B
    mesh = plsc.VectorSubcoreMesh(core_axis_name="c", subcore_axis_name="s",
                                  num_cores=NSC, num_subcores=TPS)

    def body(ex_ref, dst_ref, denp_ref, dstb, exb, exT, den_sp, sem1):
        cid = lax.axis_index("c")
        sid = lax.axis_index("s")
        wid = sid * NSC + cid
        r0 = sid * rows_pt
        lanes = lax.iota(jnp.int32, 16)
        z16 = jnp.zeros((16,), jnp.float32)

        def zex(i, _):
            exT[i, :] = z16
            return 0
        lax.fori_loop(0, B, zex, 0)
        for p in range(npieces):
            pltpu.sync_copy(exT, den_sp.at[pl.ds(r0 + p * B, B)])
        plsc.subcore_barrier()

        def block_body(bi, _):
            blk = wid + NTILES * bi

            @pl.when(blk < nblk)
            def _():
                e0 = blk * B
                pltpu.sync_copy(dst_ref.at[pl.ds(e0, B)], dstb)
                pltpu.sync_copy(ex_ref.at[pl.ds(0, h), pl.ds(e0, B)], exb)

                def exrow(g, _):
                    evs = [exb[hh, pl.ds(g * 16, 16)] for hh in range(h)]
                    for k in range(16):
                        vec = z16
                        for hh in range(h):
                            vec = jnp.where(lanes == hh, evs[hh][k], vec)
                        exT[g * 16 + k, :] = vec
                    return 0
                lax.fori_loop(0, grp, exrow, 0)
                pltpu.sync_copy(exT, den_sp.at[dstb], add=True)
            return 0
        lax.fori_loop(0, bpt, block_body, 0)
        plsc.subcore_barrier()

        for p in range(npieces):
            rr = r0 + p * B
            pltpu.sync_copy(den_sp.at[pl.ds(rr, B)], exT)
            pltpu.sync_copy(exT, denp_ref.at[cid, pl.ds(rr, B)])

    return pl.kernel(
        body,
        out_type=jax.ShapeDtypeStruct((2, n, 16), jnp.float32),
        mesh=mesh,
        compiler_params=pltpu.CompilerParams(needs_layout_passes=False),
        scratch_types=[
            pltpu.VMEM((B,), jnp.int32),
            pltpu.VMEM((h, B), jnp.float32),
            pltpu.VMEM((B, 16), jnp.float32),
            pltpu.VMEM_SHARED((n, 16), jnp.float32),
            pltpu.SemaphoreType.DMA,
        ],
    )


def _sc_aggr(n, e, h, c_pad, F, nch):
    hcp = h * c_pad
    nblk = e // B
    bps = -(-nblk // TPS)
    rows_pt = n // TPS
    grp = B // 16
    Fb = 64
    nchb = 2 * nch
    nf = Fb // 16
    npass = -(-nchb // NSC)
    npieces = rows_pt // B
    mesh = plsc.VectorSubcoreMesh(core_axis_name="c", subcore_axis_name="s",
                                  num_cores=NSC, num_subcores=TPS)

    def body(xlr_ref, exsel_ref, densel_ref, src_ref, dst_ref, bias_ref,
             y_ref, srcb, dstb, idxs, rowsb, halfb, exb1, biasv, denb1,
             out_sp, sem1):
        cid = lax.axis_index("c")
        sid = lax.axis_index("s")
        r0 = sid * rows_pt
        z16 = jnp.zeros((16,), jnp.float32)

        pltpu.sync_copy(bias_ref, biasv)

        def zrow(i, _):
            for j in range(nf):
                halfb[i, pl.ds(j * 16, 16)] = z16
            return 0

        def fpass(fp, _):
            f = cid + NSC * fp
            off = (f % 2) * Fb

            lax.fori_loop(0, B, zrow, 0)
            for p in range(npieces):
                pltpu.sync_copy(halfb, out_sp.at[pl.ds(r0 + p * B, B)])
            plsc.subcore_barrier()

            def block_body(bi, _):
                blk = sid + TPS * bi

                @pl.when(blk < nblk)
                def _():
                    e0 = blk * B
                    pltpu.sync_copy(src_ref.at[pl.ds(e0, B)], srcb)
                    pltpu.sync_copy(dst_ref.at[pl.ds(e0, B)], dstb)

                    def mkidx(j, _):
                        s = srcb[pl.ds(j * 16, 16)]
                        idxs[pl.ds(j * 16, 16)] = s * (2 * nch) + (f // 2)
                        return 0
                    lax.fori_loop(0, grp, mkidx, 0)
                    pltpu.async_copy(xlr_ref.at[idxs], rowsb, sem1).wait()
                    pltpu.sync_copy(
                        exsel_ref.at[f, 0, pl.ds(e0, B)], exb1)

                    def scale(g, _):
                        ev = exb1[pl.ds(g * 16, 16)]
                        for k in range(16):
                            a = ev[k]
                            i = g * 16 + k
                            for j in range(nf):
                                v = rowsb[i, pl.ds(off + j * 16, 16)]
                                halfb[i, pl.ds(j * 16, 16)] = v * a
                        return 0
                    lax.fori_loop(0, grp, scale, 0)
                    pltpu.sync_copy(halfb, out_sp.at[dstb], add=True)
                return 0
            lax.fori_loop(0, bps, block_body, 0)
            plsc.subcore_barrier()

            for p in range(npieces):
                rr = r0 + p * B
                pltpu.sync_copy(out_sp.at[pl.ds(rr, B)], halfb)
                pltpu.sync_copy(densel_ref.at[f, 0, pl.ds(rr, B)], denb1)
                bvs = [biasv[pl.ds(f * Fb + j * 16, 16)]
                       for j in range(nf)]

                def bgrp(g, _):
                    dvv = denb1[pl.ds(g * 16, 16)]
                    for k in range(16):
                        dv = dvv[k] + EPS
                        i = g * 16 + k
                        for j in range(nf):
                            v = halfb[i, pl.ds(j * 16, 16)] / dv + bvs[j]
                            v = jnp.where(v > 0, v, jnp.exp(v) - 1.0)
                            halfb[i, pl.ds(j * 16, 16)] = v
                    return 0
                lax.fori_loop(0, grp, bgrp, 0)
                pltpu.sync_copy(halfb, y_ref.at[f, pl.ds(rr, B)])
            plsc.subcore_barrier()
            return 0
        lax.fori_loop(0, npass, fpass, 0)

    return pl.kernel(
        body,
        out_type=jax.ShapeDtypeStruct((nchb, n, Fb), jnp.float32),
        mesh=mesh,
        compiler_params=pltpu.CompilerParams(needs_layout_passes=False),
        scratch_types=[
            pltpu.VMEM((B,), jnp.int32), pltpu.VMEM((B,), jnp.int32),
            pltpu.VMEM((B,), jnp.int32),
            pltpu.VMEM((B, F), jnp.float32),
            pltpu.VMEM((B, Fb), jnp.float32),
            pltpu.VMEM((B,), jnp.float32),
            pltpu.VMEM((hcp,), jnp.float32),
            pltpu.VMEM((B,), jnp.float32),
            pltpu.VMEM_SHARED((n, Fb), jnp.float32),
            pltpu.SemaphoreType.DMA,
        ],
    )


BN = 512


def _pool_body(ids_ref, x_ref, s_ref, c_ref):
    i = pl.program_id(0)
    ids = ids_ref[0, 0, :]
    oh = (lax.broadcasted_iota(jnp.int32, (NB, BN), 0)
          == ids[None, :]).astype(jnp.float32)
    ps = jnp.dot(oh, x_ref[...], preferred_element_type=jnp.float32)
    pc = jnp.sum(oh, axis=1)

    @pl.when(i == 0)
    def _():
        s_ref[...] = jnp.zeros_like(s_ref)
        c_ref[...] = jnp.zeros_like(c_ref)
    s_ref[...] += ps
    c_ref[...] += jnp.broadcast_to(pc[:, None], c_ref.shape)


def _pool_sums(x, batch):
    n, d = x.shape
    ids3 = batch.reshape(n // BN, 1, BN)
    return pl.pallas_call(
        _pool_body,
        grid=(n // BN,),
        in_specs=[
            pl.BlockSpec((1, 1, BN), lambda i: (i, 0, 0)),
            pl.BlockSpec((BN, d), lambda i: (i, 0)),
        ],
        out_specs=[
            pl.BlockSpec((NB, d), lambda i: (0, 0)),
            pl.BlockSpec((NB, 128), lambda i: (0, 0)),
        ],
        out_shape=[jax.ShapeDtypeStruct((NB, d), jnp.float32),
                   jax.ShapeDtypeStruct((NB, 128), jnp.float32)],
    )(ids3, x)


def _comb_body(ss_ref, sc_ref, ts_ref, tc_ref, o1_ref, o2_ref):
    x = (ss_ref[...] / jnp.maximum(sc_ref[:, 0:1], 1.0)
         + ts_ref[...] / jnp.maximum(tc_ref[:, 0:1], 1.0))
    o1_ref[...] = x
    o2_ref[...] = jax.nn.sigmoid(x)


def _combine(ss, sc, ts, tc):
    d = ss.shape[1]
    return pl.pallas_call(
        _comb_body,
        out_shape=[jax.ShapeDtypeStruct((NB, d), jnp.float32),
                   jax.ShapeDtypeStruct((NB, d), jnp.float32)],
    )(ss, sc, ts, tc)


def _pad_cols(w, tgt):
    return jnp.pad(w, ((0, 0), (0, tgt - w.shape[1])))


def _gat_layer(x, src, dst, ea, Wl, bl, Wr, br, We, att, bias):
    n = x.shape[0]
    e = src.shape[0]
    h, c = att.shape
    if h == 4:
        c_pad, F, nch = 256, 128, 8
    else:
        c_pad, F, nch = 1408, 128, 11
    hcp = h * c_pad

    Wlp = _pad_cols(Wl, hcp)
    Wrp = _pad_cols(Wr, hcp)
    W2 = jnp.concatenate([Wlp, Wrp], axis=1)
    b2 = jnp.concatenate([
        jnp.pad(bl, (0, hcp - bl.shape[0])),
        jnp.pad(br, (0, hcp - br.shape[0])),
    ])
    xlr = _matmul_bias(x, W2, b2)
    xlr_r = xlr.reshape(n * 2 * nch, F)

    ea_p = jnp.pad(ea, ((0, 0), (0, 16 - ea.shape[1])))
    We_p = _pad_cols(jnp.pad(We, ((0, 16 - We.shape[0]), (0, 0))), hcp)
    ee = _matmul_bias(ea_p, We_p, jnp.zeros((hcp,), jnp.float32), bm=2048)
    ee_r = ee.reshape(e * nch, F)

    att_flat = jnp.pad(att.reshape(-1), (0, hcp - h * c))
    bias_p = jnp.pad(bias, (0, hcp - bias.shape[0]))

    ex = _sc_attn(n, e, h, c_pad, F, nch)(xlr_r, ee_r, src, dst, att_flat)
    denP = _sc_den(n, e, h)(ex, dst)
    nchb = 2 * nch
    head_map = jnp.array([(f * 64) // c_pad for f in range(nchb)],
                         dtype=jnp.int32)
    exsel = ex[head_map].reshape(nchb, 1, e)
    den = (denP[0] + denP[1])[:, :h]
    densel = den.T[head_map].reshape(nchb, 1, n)
    y3 = _sc_aggr(n, e, h, c_pad, F, nch)(
        xlr_r, exsel, densel, src, dst, bias_p)
    return y3.transpose(1, 0, 2).reshape(n, hcp)


def _branch(x, ei, ea, params):
    src, dst = ei[0], ei[1]
    for (Wl, bl, Wr, br, We, att, bias) in params:
        x = _gat_layer(x, src, dst, ea, Wl, bl, Wr, br, We, att, bias)
    return x


def kernel(x_s, edge_index_s, edge_attr_s, x_t, edge_index_t, edge_attr_t, xs_batch, xt_batch, s1_Wl, s1_bl, s1_Wr, s1_br, s1_We, s1_att, s1_bias, s2_Wl, s2_bl, s2_Wr, s2_br, s2_We, s2_att, s2_bias, s3_Wl, s3_bl, s3_Wr, s3_br, s3_We, s3_att, s3_bias, t1_Wl, t1_bl, t1_Wr, t1_br, t1_We, t1_att, t1_bias, t2_Wl, t2_bl, t2_Wr, t2_br, t2_We, t2_att, t2_bias, t3_Wl, t3_bl, t3_Wr, t3_br, t3_We, t3_att, t3_bias):
    ps = [
        (s1_Wl, s1_bl, s1_Wr, s1_br, s1_We, s1_att, s1_bias),
        (s2_Wl, s2_bl, s2_Wr, s2_br, s2_We, s2_att, s2_bias),
        (s3_Wl, s3_bl, s3_Wr, s3_br, s3_We, s3_att, s3_bias),
    ]
    pt = [
        (t1_Wl, t1_bl, t1_Wr, t1_br, t1_We, t1_att, t1_bias),
        (t2_Wl, t2_bl, t2_Wr, t2_br, t2_We, t2_att, t2_bias),
        (t3_Wl, t3_bl, t3_Wr, t3_br, t3_We, t3_att, t3_bias),
    ]
    npad = 240
    x_s = jnp.pad(x_s, ((0, npad), (0, 0)))
    x_t = jnp.pad(x_t, ((0, npad), (0, 0)))
    xs_batch = jnp.pad(xs_batch, (0, npad), constant_values=NB)
    xt_batch = jnp.pad(xt_batch, (0, npad), constant_values=NB)
    xs = _branch(x_s, edge_index_s, edge_attr_s, ps)
    xt = _branch(x_t, edge_index_t, edge_attr_t, pt)
    ss, sc = _pool_sums(xs, xs_batch)
    ts, tc = _pool_sums(xt, xt_batch)
    x, sg = _combine(ss, sc, ts, tc)
    return (x[:, :OUT], sg[:, :OUT])


# speedup vs baseline: 1.1648x; 1.0001x over previous
import jax
import jax.numpy as jnp
from jax import lax
from jax.experimental import pallas as pl
from jax.experimental.pallas import tpu as pltpu
from jax.experimental.pallas import tpu_sc as plsc

NB = 64
OUT = 1317
B = 128
NSC = 2
TPS = 16
NTILES = NSC * TPS
EPS = 1e-16


def _mm_body(x_ref, w_ref, b_ref, o_ref):
    o_ref[...] = (
        jnp.dot(x_ref[...], w_ref[...], preferred_element_type=jnp.float32)
        + b_ref[...]
    )


def _matmul_bias(x, w, b, bm=512):
    m, k = x.shape
    _, n = w.shape
    return pl.pallas_call(
        _mm_body,
        grid=(pl.cdiv(m, bm),),
        in_specs=[
            pl.BlockSpec((bm, k), lambda i: (i, 0)),
            pl.BlockSpec((k, n), lambda i: (0, 0)),
            pl.BlockSpec((1, n), lambda i: (0, 0)),
        ],
        out_specs=pl.BlockSpec((bm, n), lambda i: (i, 0)),
        out_shape=jax.ShapeDtypeStruct((m, n), jnp.float32),
    )(x, w, b.reshape(1, n))


def _sc_attn(n, e, h, c_pad, F, nch):
    hcp = h * c_pad
    nblk = e // B
    bpt = -(-nblk // NTILES)
    grp = B // 16
    nf = F // 16
    mesh = plsc.VectorSubcoreMesh(core_axis_name="c", subcore_axis_name="s",
                                  num_cores=NSC, num_subcores=TPS)

    def body(xlr_ref, ee_ref, src_ref, dst_ref, att_ref, ex_ref,
             srcb, dstb, idxs0, idxd0, idxe0, idxs1, idxd1, idxe1,
             xlb0, xrb0, eeb0, xlb1, xrb1, eeb1, logb, exb, attv,
             s10, s20, s30, s11, s21, s31):
        cid = lax.axis_index("c")
        sid = lax.axis_index("s")
        wid = sid * NSC + cid
        lanes = lax.iota(jnp.int32, 16)
        z16 = jnp.zeros((16,), jnp.float32)

        idxbufs = [(idxs0, idxd0, idxe0), (idxs1, idxd1, idxe1)]
        rowbufs = [(xlb0, xrb0, eeb0), (xlb1, xrb1, eeb1)]
        sems = [(s10, s20, s30), (s11, s21, s31)]

        pltpu.sync_copy(att_ref, attv)

        def block_body(bi, _):
            blk = wid + NTILES * bi

            @pl.when(blk < nblk)
            def _():
                e0 = blk * B
                pltpu.sync_copy(src_ref.at[pl.ds(e0, B)], srcb)
                pltpu.sync_copy(dst_ref.at[pl.ds(e0, B)], dstb)
                for hh in range(h):
                    def zrow(g, _):
                        logb[hh, pl.ds(g * 16, 16)] = z16
                        return 0
                    lax.fori_loop(0, grp, zrow, 0)

                def issue(f):
                    b = f % 2
                    idxs, idxd, idxe = idxbufs[b]
                    xlb, xrb, eeb = rowbufs[b]
                    sa, sb, sc = sems[b]

                    def mkidx(j, _):
                        s = srcb[pl.ds(j * 16, 16)]
                        idxs[pl.ds(j * 16, 16)] = s * (2 * nch) + f
                        d = dstb[pl.ds(j * 16, 16)]
                        idxd[pl.ds(j * 16, 16)] = d * (2 * nch) + (nch + f)
                        idxe[pl.ds(j * 16, 16)] = (
                            (e0 + j * 16 + lanes) * nch + f)
                        return 0
                    lax.fori_loop(0, grp, mkidx, 0)
                    return (pltpu.async_copy(xlr_ref.at[idxs], xlb, sa),
                            pltpu.async_copy(xlr_ref.at[idxd], xrb, sb),
                            pltpu.async_copy(ee_ref.at[idxe], eeb, sc))

                cps = issue(0)
                for f in range(nch):
                    h0 = (f * F) // c_pad
                    xlb, xrb, eeb = rowbufs[f % 2]
                    for cp in cps:
                        cp.wait()
                    if f + 1 < nch:
                        cps = issue(f + 1)

                    def grpbody(g, _):
                        def edge(k, vec):
                            i = g * 16 + k
                            acc = z16
                            for j in range(nf):
                                cs = pl.ds(j * 16, 16)
                                z = xlb[i, cs] + xrb[i, cs] + eeb[i, cs]
                                z = jnp.maximum(z, z * 0.2)
                                acc = acc + z * attv[
                                    pl.ds(f * F + j * 16, 16)]
                            s = jnp.sum(acc)
                            return jnp.where(lanes == k, s, vec)
                        vec = lax.fori_loop(0, 16, edge, z16)
                        cur = logb[h0, pl.ds(g * 16, 16)]
                        logb[h0, pl.ds(g * 16, 16)] = cur + vec
                        return 0
                    lax.fori_loop(0, grp, grpbody, 0)

                def p2(g, _):
                    for hh in range(h):
                        ev = jnp.exp(logb[hh, pl.ds(g * 16, 16)])
                        exb[hh, pl.ds(g * 16, 16)] = ev
                    return 0
                lax.fori_loop(0, grp, p2, 0)
                pltpu.sync_copy(exb, ex_ref.at[pl.ds(0, h), pl.ds(e0, B)])
            return 0
        lax.fori_loop(0, bpt, block_body, 0)

    return pl.kernel(
        body,
        out_type=jax.ShapeDtypeStruct((h, e), jnp.float32),
        mesh=mesh,
        compiler_params=pltpu.CompilerParams(needs_layout_passes=False),
        scratch_types=(
            [pltpu.VMEM((B,), jnp.int32)] * 2
            + [pltpu.VMEM((B,), jnp.int32)] * 6
            + [pltpu.VMEM((B, F), jnp.float32)] * 6
            + [pltpu.VMEM((h, B), jnp.float32)] * 2
            + [pltpu.VMEM((hcp,), jnp.float32)]
            + [pltpu.SemaphoreType.DMA] * 6
        ),
    )


def _sc_den(n, e, h):
    nblk = e // B
    bpt = -(-nblk // NTILES)
    rows_pt = n // TPS
    grp = B // 16
    npieces = rows_pt // B
    mesh = plsc.VectorSubcoreMesh(core_axis_name="c", subcore_axis_name="s",
                                  num_cores=NSC, num_subcores=TPS)

    def body(ex_ref, dst_ref, denp_ref, dstb, exb, exT, den_sp, sem1):
        cid = lax.axis_index("c")
        sid = lax.axis_index("s")
        wid = sid * NSC + cid
        r0 = sid * rows_pt
        lanes = lax.iota(jnp.int32, 16)
        z16 = jnp.zeros((16,), jnp.float32)

        def zex(i, _):
            exT[i, :] = z16
            return 0
        lax.fori_loop(0, B, zex, 0)
        for p in range(npieces):
            pltpu.sync_copy(exT, den_sp.at[pl.ds(r0 + p * B, B)])
        plsc.subcore_barrier()

        def block_body(bi, _):
            blk = wid + NTILES * bi

            @pl.when(blk < nblk)
            def _():
                e0 = blk * B
                pltpu.sync_copy(dst_ref.at[pl.ds(e0, B)], dstb)
                pltpu.sync_copy(ex_ref.at[pl.ds(0, h), pl.ds(e0, B)], exb)

                def exrow(g, _):
                    evs = [exb[hh, pl.ds(g * 16, 16)] for hh in range(h)]
                    for k in range(16):
                        vec = z16
                        for hh in range(h):
                            vec = jnp.where(lanes == hh, evs[hh][k], vec)
                        exT[g * 16 + k, :] = vec
                    return 0
                lax.fori_loop(0, grp, exrow, 0)
                pltpu.sync_copy(exT, den_sp.at[dstb], add=True)
            return 0
        lax.fori_loop(0, bpt, block_body, 0)
        plsc.subcore_barrier()

        for p in range(npieces):
            rr = r0 + p * B
            pltpu.sync_copy(den_sp.at[pl.ds(rr, B)], exT)
            pltpu.sync_copy(exT, denp_ref.at[cid, pl.ds(rr, B)])

    return pl.kernel(
        body,
        out_type=jax.ShapeDtypeStruct((2, n, 16), jnp.float32),
        mesh=mesh,
        compiler_params=pltpu.CompilerParams(needs_layout_passes=False),
        scratch_types=[
            pltpu.VMEM((B,), jnp.int32),
            pltpu.VMEM((h, B), jnp.float32),
            pltpu.VMEM((B, 16), jnp.float32),
            pltpu.VMEM_SHARED((n, 16), jnp.float32),
            pltpu.SemaphoreType.DMA,
        ],
    )


def _sc_aggr(n, e, h, c_pad, F, nch):
    hcp = h * c_pad
    nblk = e // B
    bps = -(-nblk // TPS)
    rows_pt = n // TPS
    grp = B // 16
    Fb = 64
    nchb = 2 * nch
    nf = Fb // 16
    npass = -(-nchb // NSC)
    npieces = rows_pt // B
    mesh = plsc.VectorSubcoreMesh(core_axis_name="c", subcore_axis_name="s",
                                  num_cores=NSC, num_subcores=TPS)

    def body(xlr_ref, exsel_ref, densel_ref, src_ref, dst_ref, bias_ref,
             y_ref, srcb, dstb, idxs, rowsb, srcb2, dstb2, idxs2, rowsb2,
             halfb, exb1, biasv, denb1, out_sp, sem1, sem2):
        cid = lax.axis_index("c")
        sid = lax.axis_index("s")
        r0 = sid * rows_pt
        z16 = jnp.zeros((16,), jnp.float32)

        pltpu.sync_copy(bias_ref, biasv)

        def zrow(i, _):
            for j in range(nf):
                halfb[i, pl.ds(j * 16, 16)] = z16
            return 0

        def fpass(fp, _):
            f = cid + NSC * fp
            off = (f % 2) * Fb

            lax.fori_loop(0, B, zrow, 0)
            for p in range(npieces):
                pltpu.sync_copy(halfb, out_sp.at[pl.ds(r0 + p * B, B)])
            plsc.subcore_barrier()

            def block_body(bi, _):
                blk = sid + TPS * bi

                @pl.when(blk < nblk)
                def _():
                    e0 = blk * B
                    pltpu.sync_copy(src_ref.at[pl.ds(e0, B)], srcb)
                    pltpu.sync_copy(dst_ref.at[pl.ds(e0, B)], dstb)

                    def mkidx(j, _):
                        s = srcb[pl.ds(j * 16, 16)]
                        idxs[pl.ds(j * 16, 16)] = s * (2 * nch) + (f // 2)
                        return 0
                    lax.fori_loop(0, grp, mkidx, 0)
                    pltpu.async_copy(xlr_ref.at[idxs], rowsb, sem1).wait()
                    pltpu.sync_copy(
                        exsel_ref.at[f, 0, pl.ds(e0, B)], exb1)

                    def scale(g, _):
                        ev = exb1[pl.ds(g * 16, 16)]
                        for k in range(16):
                            a = ev[k]
                            i = g * 16 + k
                            for j in range(nf):
                                v = rowsb[i, pl.ds(off + j * 16, 16)]
                                halfb[i, pl.ds(j * 16, 16)] = v * a
                        return 0
                    lax.fori_loop(0, grp, scale, 0)
                    pltpu.sync_copy(halfb, out_sp.at[dstb], add=True)
                return 0
            lax.fori_loop(0, bps, block_body, 0)
            plsc.subcore_barrier()

            for p in range(npieces):
                rr = r0 + p * B
                pltpu.sync_copy(out_sp.at[pl.ds(rr, B)], halfb)
                pltpu.sync_copy(densel_ref.at[f, 0, pl.ds(rr, B)], denb1)
                bvs = [biasv[pl.ds(f * Fb + j * 16, 16)]
                       for j in range(nf)]

                def bgrp(g, _):
                    dvv = denb1[pl.ds(g * 16, 16)]
                    for k in range(16):
                        dv = dvv[k] + EPS
                        i = g * 16 + k
                        for j in range(nf):
                            v = halfb[i, pl.ds(j * 16, 16)] / dv + bvs[j]
                            v = jnp.where(v > 0, v, jnp.exp(v) - 1.0)
                            halfb[i, pl.ds(j * 16, 16)] = v
                    return 0
                lax.fori_loop(0, grp, bgrp, 0)
                pltpu.sync_copy(halfb, y_ref.at[f, pl.ds(rr, B)])
            plsc.subcore_barrier()
            return 0
        lax.fori_loop(0, npass, fpass, 0)

    return pl.kernel(
        body,
        out_type=jax.ShapeDtypeStruct((nchb, n, Fb), jnp.float32),
        mesh=mesh,
        compiler_params=pltpu.CompilerParams(needs_layout_passes=False),
        scratch_types=[
            pltpu.VMEM((B,), jnp.int32), pltpu.VMEM((B,), jnp.int32),
            pltpu.VMEM((B,), jnp.int32),
            pltpu.VMEM((B, F), jnp.float32),
            pltpu.VMEM((B,), jnp.int32), pltpu.VMEM((B,), jnp.int32),
            pltpu.VMEM((B,), jnp.int32),
            pltpu.VMEM((B, F), jnp.float32),
            pltpu.VMEM((B, Fb), jnp.float32),
            pltpu.VMEM((B,), jnp.float32),
            pltpu.VMEM((hcp,), jnp.float32),
            pltpu.VMEM((B,), jnp.float32),
            pltpu.VMEM_SHARED((n, Fb), jnp.float32),
            pltpu.SemaphoreType.DMA, pltpu.SemaphoreType.DMA,
        ],
    )


BN = 512


def _pool_body(ids_ref, x_ref, s_ref, c_ref):
    i = pl.program_id(0)
    ids = ids_ref[0, 0, :]
    oh = (lax.broadcasted_iota(jnp.int32, (NB, BN), 0)
          == ids[None, :]).astype(jnp.float32)
    ps = jnp.dot(oh, x_ref[...], preferred_element_type=jnp.float32)
    pc = jnp.sum(oh, axis=1)

    @pl.when(i == 0)
    def _():
        s_ref[...] = jnp.zeros_like(s_ref)
        c_ref[...] = jnp.zeros_like(c_ref)
    s_ref[...] += ps
    c_ref[...] += jnp.broadcast_to(pc[:, None], c_ref.shape)


def _pool_sums(x, batch):
    n, d = x.shape
    ids3 = batch.reshape(n // BN, 1, BN)
    return pl.pallas_call(
        _pool_body,
        grid=(n // BN,),
        in_specs=[
            pl.BlockSpec((1, 1, BN), lambda i: (i, 0, 0)),
            pl.BlockSpec((BN, d), lambda i: (i, 0)),
        ],
        out_specs=[
            pl.BlockSpec((NB, d), lambda i: (0, 0)),
            pl.BlockSpec((NB, 128), lambda i: (0, 0)),
        ],
        out_shape=[jax.ShapeDtypeStruct((NB, d), jnp.float32),
                   jax.ShapeDtypeStruct((NB, 128), jnp.float32)],
    )(ids3, x)


def _comb_body(ss_ref, sc_ref, ts_ref, tc_ref, o1_ref, o2_ref):
    x = (ss_ref[...] / jnp.maximum(sc_ref[:, 0:1], 1.0)
         + ts_ref[...] / jnp.maximum(tc_ref[:, 0:1], 1.0))
    o1_ref[...] = x
    o2_ref[...] = jax.nn.sigmoid(x)


def _combine(ss, sc, ts, tc):
    d = ss.shape[1]
    return pl.pallas_call(
        _comb_body,
        out_shape=[jax.ShapeDtypeStruct((NB, d), jnp.float32),
                   jax.ShapeDtypeStruct((NB, d), jnp.float32)],
    )(ss, sc, ts, tc)


def _pad_cols(w, tgt):
    return jnp.pad(w, ((0, 0), (0, tgt - w.shape[1])))


def _gat_layer(x, src, dst, ea, Wl, bl, Wr, br, We, att, bias):
    n = x.shape[0]
    e = src.shape[0]
    h, c = att.shape
    if h == 4:
        c_pad, F, nch = 256, 128, 8
    else:
        c_pad, F, nch = 1408, 128, 11
    hcp = h * c_pad

    Wlp = _pad_cols(Wl, hcp)
    Wrp = _pad_cols(Wr, hcp)
    W2 = jnp.concatenate([Wlp, Wrp], axis=1)
    b2 = jnp.concatenate([
        jnp.pad(bl, (0, hcp - bl.shape[0])),
        jnp.pad(br, (0, hcp - br.shape[0])),
    ])
    xlr = _matmul_bias(x, W2, b2)
    xlr_r = xlr.reshape(n * 2 * nch, F)

    ea_p = jnp.pad(ea, ((0, 0), (0, 16 - ea.shape[1])))
    We_p = _pad_cols(jnp.pad(We, ((0, 16 - We.shape[0]), (0, 0))), hcp)
    ee = _matmul_bias(ea_p, We_p, jnp.zeros((hcp,), jnp.float32), bm=2048)
    ee_r = ee.reshape(e * nch, F)

    att_flat = jnp.pad(att.reshape(-1), (0, hcp - h * c))
    bias_p = jnp.pad(bias, (0, hcp - bias.shape[0]))

    ex = _sc_attn(n, e, h, c_pad, F, nch)(xlr_r, ee_r, src, dst, att_flat)
    denP = _sc_den(n, e, h)(ex, dst)
    nchb = 2 * nch
    head_map = jnp.array([(f * 64) // c_pad for f in range(nchb)],
                         dtype=jnp.int32)
    exsel = ex[head_map].reshape(nchb, 1, e)
    den = (denP[0] + denP[1])[:, :h]
    densel = den.T[head_map].reshape(nchb, 1, n)
    y3 = _sc_aggr(n, e, h, c_pad, F, nch)(
        xlr_r, exsel, densel, src, dst, bias_p)
    return y3.transpose(1, 0, 2).reshape(n, hcp)


def _branch(x, ei, ea, params):
    src, dst = ei[0], ei[1]
    for (Wl, bl, Wr, br, We, att, bias) in params:
        x = _gat_layer(x, src, dst, ea, Wl, bl, Wr, br, We, att, bias)
    return x


def kernel(x_s, edge_index_s, edge_attr_s, x_t, edge_index_t, edge_attr_t, xs_batch, xt_batch, s1_Wl, s1_bl, s1_Wr, s1_br, s1_We, s1_att, s1_bias, s2_Wl, s2_bl, s2_Wr, s2_br, s2_We, s2_att, s2_bias, s3_Wl, s3_bl, s3_Wr, s3_br, s3_We, s3_att, s3_bias, t1_Wl, t1_bl, t1_Wr, t1_br, t1_We, t1_att, t1_bias, t2_Wl, t2_bl, t2_Wr, t2_br, t2_We, t2_att, t2_bias, t3_Wl, t3_bl, t3_Wr, t3_br, t3_We, t3_att, t3_bias):
    ps = [
        (s1_Wl, s1_bl, s1_Wr, s1_br, s1_We, s1_att, s1_bias),
        (s2_Wl, s2_bl, s2_Wr, s2_br, s2_We, s2_att, s2_bias),
        (s3_Wl, s3_bl, s3_Wr, s3_br, s3_We, s3_att, s3_bias),
    ]
    pt = [
        (t1_Wl, t1_bl, t1_Wr, t1_br, t1_We, t1_att, t1_bias),
        (t2_Wl, t2_bl, t2_Wr, t2_br, t2_We, t2_att, t2_bias),
        (t3_Wl, t3_bl, t3_Wr, t3_br, t3_We, t3_att, t3_bias),
    ]
    npad = 240
    x_s = jnp.pad(x_s, ((0, npad), (0, 0)))
    x_t = jnp.pad(x_t, ((0, npad), (0, 0)))
    xs_batch = jnp.pad(xs_batch, (0, npad), constant_values=NB)
    xt_batch = jnp.pad(xt_batch, (0, npad), constant_values=NB)
    xs = _branch(x_s, edge_index_s, edge_attr_s, ps)
    xt = _branch(x_t, edge_index_t, edge_attr_t, pt)
    ss, sc = _pool_sums(xs, xs_batch)
    ts, tc = _pool_sums(xt, xt_batch)
    x, sg = _combine(ss, sc, ts, tc)
    return (x[:, :OUT], sg[:, :OUT])


# speedup vs baseline: 1.1652x; 1.0003x over previous
import jax
import jax.numpy as jnp
from jax import lax
from jax.experimental import pallas as pl
from jax.experimental.pallas import tpu as pltpu
from jax.experimental.pallas import tpu_sc as plsc

NB = 64
OUT = 1317
B = 128
NSC = 2
TPS = 16
NTILES = NSC * TPS
EPS = 1e-16


def _mm_body(x_ref, w_ref, b_ref, o_ref):
    o_ref[...] = (
        jnp.dot(x_ref[...], w_ref[...], preferred_element_type=jnp.float32)
        + b_ref[...]
    )


def _matmul_bias(x, w, b, bm=512):
    m, k = x.shape
    _, n = w.shape
    return pl.pallas_call(
        _mm_body,
        grid=(pl.cdiv(m, bm),),
        in_specs=[
            pl.BlockSpec((bm, k), lambda i: (i, 0)),
            pl.BlockSpec((k, n), lambda i: (0, 0)),
            pl.BlockSpec((1, n), lambda i: (0, 0)),
        ],
        out_specs=pl.BlockSpec((bm, n), lambda i: (i, 0)),
        out_shape=jax.ShapeDtypeStruct((m, n), jnp.float32),
    )(x, w, b.reshape(1, n))


def _sc_attn(n, e, h, c_pad, F, nch):
    hcp = h * c_pad
    nblk = e // B
    bpt = -(-nblk // NTILES)
    grp = B // 16
    nf = F // 16
    mesh = plsc.VectorSubcoreMesh(core_axis_name="c", subcore_axis_name="s",
                                  num_cores=NSC, num_subcores=TPS)

    def body(xlr_ref, ee_ref, src_ref, dst_ref, att_ref, ex_ref,
             srcb, dstb, idxs0, idxd0, idxe0, idxs1, idxd1, idxe1,
             xlb0, xrb0, eeb0, xlb1, xrb1, eeb1, logb, exb, attv,
             s10, s20, s30, s11, s21, s31):
        cid = lax.axis_index("c")
        sid = lax.axis_index("s")
        wid = sid * NSC + cid
        lanes = lax.iota(jnp.int32, 16)
        z16 = jnp.zeros((16,), jnp.float32)

        idxbufs = [(idxs0, idxd0, idxe0), (idxs1, idxd1, idxe1)]
        rowbufs = [(xlb0, xrb0, eeb0), (xlb1, xrb1, eeb1)]
        sems = [(s10, s20, s30), (s11, s21, s31)]

        pltpu.sync_copy(att_ref, attv)

        def block_body(bi, _):
            blk = wid + NTILES * bi

            @pl.when(blk < nblk)
            def _():
                e0 = blk * B
                pltpu.sync_copy(src_ref.at[pl.ds(e0, B)], srcb)
                pltpu.sync_copy(dst_ref.at[pl.ds(e0, B)], dstb)
                for hh in range(h):
                    def zrow(g, _):
                        logb[hh, pl.ds(g * 16, 16)] = z16
                        return 0
                    lax.fori_loop(0, grp, zrow, 0)

                def issue(f):
                    b = f % 2
                    idxs, idxd, idxe = idxbufs[b]
                    xlb, xrb, eeb = rowbufs[b]
                    sa, sb, sc = sems[b]

                    def mkidx(j, _):
                        s = srcb[pl.ds(j * 16, 16)]
                        idxs[pl.ds(j * 16, 16)] = s * (2 * nch) + f
                        d = dstb[pl.ds(j * 16, 16)]
                        idxd[pl.ds(j * 16, 16)] = d * (2 * nch) + (nch + f)
                        idxe[pl.ds(j * 16, 16)] = (
                            (e0 + j * 16 + lanes) * nch + f)
                        return 0
                    lax.fori_loop(0, grp, mkidx, 0)
                    return (pltpu.async_copy(xlr_ref.at[idxs], xlb, sa),
                            pltpu.async_copy(xlr_ref.at[idxd], xrb, sb),
                            pltpu.async_copy(ee_ref.at[idxe], eeb, sc))

                cps = issue(0)
                for f in range(nch):
                    h0 = (f * F) // c_pad
                    xlb, xrb, eeb = rowbufs[f % 2]
                    for cp in cps:
                        cp.wait()
                    if f + 1 < nch:
                        cps = issue(f + 1)

                    def grpbody(g, _):
                        def edge(k, vec):
                            i = g * 16 + k
                            acc = z16
                            for j in range(nf):
                                cs = pl.ds(j * 16, 16)
                                z = xlb[i, cs] + xrb[i, cs] + eeb[i, cs]
                                z = jnp.maximum(z, z * 0.2)
                                acc = acc + z * attv[
                                    pl.ds(f * F + j * 16, 16)]
                            s = jnp.sum(acc)
                            return jnp.where(lanes == k, s, vec)
                        vec = lax.fori_loop(0, 16, edge, z16)
                        cur = logb[h0, pl.ds(g * 16, 16)]
                        logb[h0, pl.ds(g * 16, 16)] = cur + vec
                        return 0
                    lax.fori_loop(0, grp, grpbody, 0)

                def p2(g, _):
                    for hh in range(h):
                        ev = jnp.exp(logb[hh, pl.ds(g * 16, 16)])
                        exb[hh, pl.ds(g * 16, 16)] = ev
                    return 0
                lax.fori_loop(0, grp, p2, 0)
                pltpu.sync_copy(exb, ex_ref.at[pl.ds(0, h), pl.ds(e0, B)])
            return 0
        lax.fori_loop(0, bpt, block_body, 0)

    return pl.kernel(
        body,
        out_type=jax.ShapeDtypeStruct((h, e), jnp.float32),
        mesh=mesh,
        compiler_params=pltpu.CompilerParams(needs_layout_passes=False),
        scratch_types=(
            [pltpu.VMEM((B,), jnp.int32)] * 2
            + [pltpu.VMEM((B,), jnp.int32)] * 6
            + [pltpu.VMEM((B, F), jnp.float32)] * 6
            + [pltpu.VMEM((h, B), jnp.float32)] * 2
            + [pltpu.VMEM((hcp,), jnp.float32)]
            + [pltpu.SemaphoreType.DMA] * 6
        ),
    )


def _sc_den(n, e, h):
    nblk = e // B
    bpt = -(-nblk // NTILES)
    rows_pt = n // TPS
    grp = B // 16
    npieces = rows_pt // B
    mesh = plsc.VectorSubcoreMesh(core_axis_name="c", subcore_axis_name="s",
                                  num_cores=NSC, num_subcores=TPS)

    def body(ex_ref, dst_ref, denp_ref, dstb, exb, exT, den_sp, sem1):
        cid = lax.axis_index("c")
        sid = lax.axis_index("s")
        wid = sid * NSC + cid
        r0 = sid * rows_pt
        lanes = lax.iota(jnp.int32, 16)
        z16 = jnp.zeros((16,), jnp.float32)

        def zex(i, _):
            exT[i, :] = z16
            return 0
        lax.fori_loop(0, B, zex, 0)
        for p in range(npieces):
            pltpu.sync_copy(exT, den_sp.at[pl.ds(r0 + p * B, B)])
        plsc.subcore_barrier()

        def block_body(bi, _):
            blk = wid + NTILES * bi

            @pl.when(blk < nblk)
            def _():
                e0 = blk * B
                pltpu.sync_copy(dst_ref.at[pl.ds(e0, B)], dstb)
                pltpu.sync_copy(ex_ref.at[pl.ds(0, h), pl.ds(e0, B)], exb)

                def exrow(g, _):
                    evs = [exb[hh, pl.ds(g * 16, 16)] for hh in range(h)]
                    for k in range(16):
                        vec = z16
                        for hh in range(h):
                            vec = jnp.where(lanes == hh, evs[hh][k], vec)
                        exT[g * 16 + k, :] = vec
                    return 0
                lax.fori_loop(0, grp, exrow, 0)
                pltpu.sync_copy(exT, den_sp.at[dstb], add=True)
            return 0
        lax.fori_loop(0, bpt, block_body, 0)
        plsc.subcore_barrier()

        for p in range(npieces):
            rr = r0 + p * B
            pltpu.sync_copy(den_sp.at[pl.ds(rr, B)], exT)
            pltpu.sync_copy(exT, denp_ref.at[cid, pl.ds(rr, B)])

    return pl.kernel(
        body,
        out_type=jax.ShapeDtypeStruct((2, n, 16), jnp.float32),
        mesh=mesh,
        compiler_params=pltpu.CompilerParams(needs_layout_passes=False),
        scratch_types=[
            pltpu.VMEM((B,), jnp.int32),
            pltpu.VMEM((h, B), jnp.float32),
            pltpu.VMEM((B, 16), jnp.float32),
            pltpu.VMEM_SHARED((n, 16), jnp.float32),
            pltpu.SemaphoreType.DMA,
        ],
    )


def _sc_aggr(n, e, h, c_pad, F, nch):
    hcp = h * c_pad
    nblk = e // B
    bps = -(-nblk // TPS)
    rows_pt = n // TPS
    grp = B // 16
    Fb = 64
    nchb = 2 * nch
    nf = Fb // 16
    npass = -(-nchb // NSC)
    npieces = rows_pt // B
    mesh = plsc.VectorSubcoreMesh(core_axis_name="c", subcore_axis_name="s",
                                  num_cores=NSC, num_subcores=TPS)

    def body(xlr_ref, exsel_ref, densel_ref, src_ref, dst_ref, bias_ref,
             y_ref, srcb, dstb, idxs, rowsb,
             halfb, exb1, biasv, denb1, out_sp, sem1):
        cid = lax.axis_index("c")
        sid = lax.axis_index("s")
        r0 = sid * rows_pt
        z16 = jnp.zeros((16,), jnp.float32)

        pltpu.sync_copy(bias_ref, biasv)

        def zrow(i, _):
            for j in range(nf):
                halfb[i, pl.ds(j * 16, 16)] = z16
            return 0

        def fpass(fp, _):
            f = cid + NSC * fp
            off = (f % 2) * Fb

            lax.fori_loop(0, B, zrow, 0)
            for p in range(npieces):
                pltpu.sync_copy(halfb, out_sp.at[pl.ds(r0 + p * B, B)])
            plsc.subcore_barrier()

            def block_body(bi, _):
                blk = sid + TPS * bi

                @pl.when(blk < nblk)
                def _():
                    e0 = blk * B
                    pltpu.sync_copy(src_ref.at[pl.ds(e0, B)], srcb)
                    pltpu.sync_copy(dst_ref.at[pl.ds(e0, B)], dstb)

                    def mkidx(j, _):
                        s = srcb[pl.ds(j * 16, 16)]
                        idxs[pl.ds(j * 16, 16)] = s * (2 * nch) + (f // 2)
                        return 0
                    lax.fori_loop(0, grp, mkidx, 0)
                    pltpu.async_copy(xlr_ref.at[idxs], rowsb, sem1).wait()
                    pltpu.sync_copy(
                        exsel_ref.at[f, 0, pl.ds(e0, B)], exb1)

                    def scale(g, _):
                        ev = exb1[pl.ds(g * 16, 16)]
                        for k in range(16):
                            a = ev[k]
                            i = g * 16 + k
                            for j in range(nf):
                                v = rowsb[i, pl.ds(off + j * 16, 16)]
                                halfb[i, pl.ds(j * 16, 16)] = v * a
                        return 0
                    lax.fori_loop(0, grp, scale, 0)
                    pltpu.sync_copy(halfb, out_sp.at[dstb], add=True)
                return 0
            lax.fori_loop(0, bps, block_body, 0)
            plsc.subcore_barrier()

            for p in range(npieces):
                rr = r0 + p * B
                pltpu.sync_copy(out_sp.at[pl.ds(rr, B)], halfb)
                pltpu.sync_copy(densel_ref.at[f, 0, pl.ds(rr, B)], denb1)
                bvs = [biasv[pl.ds(f * Fb + j * 16, 16)]
                       for j in range(nf)]

                def bgrp(g, _):
                    dvv = denb1[pl.ds(g * 16, 16)]
                    for k in range(16):
                        dv = dvv[k] + EPS
                        i = g * 16 + k
                        for j in range(nf):
                            v = halfb[i, pl.ds(j * 16, 16)] / dv + bvs[j]
                            v = jnp.where(v > 0, v, jnp.exp(v) - 1.0)
                            halfb[i, pl.ds(j * 16, 16)] = v
                    return 0
                lax.fori_loop(0, grp, bgrp, 0)
                pltpu.sync_copy(halfb, y_ref.at[f, pl.ds(rr, B)])
            plsc.subcore_barrier()
            return 0
        lax.fori_loop(0, npass, fpass, 0)

    return pl.kernel(
        body,
        out_type=jax.ShapeDtypeStruct((nchb, n, Fb), jnp.float32),
        mesh=mesh,
        compiler_params=pltpu.CompilerParams(needs_layout_passes=False),
        scratch_types=[
            pltpu.VMEM((B,), jnp.int32), pltpu.VMEM((B,), jnp.int32),
            pltpu.VMEM((B,), jnp.int32),
            pltpu.VMEM((B, F), jnp.float32),
            pltpu.VMEM((B, Fb), jnp.float32),
            pltpu.VMEM((B,), jnp.float32),
            pltpu.VMEM((hcp,), jnp.float32),
            pltpu.VMEM((B,), jnp.float32),
            pltpu.VMEM_SHARED((n, Fb), jnp.float32),
            pltpu.SemaphoreType.DMA,
        ],
    )


BN = 512


def _pool_body(ids_ref, x_ref, s_ref, c_ref):
    i = pl.program_id(0)
    ids = ids_ref[0, 0, :]
    oh = (lax.broadcasted_iota(jnp.int32, (NB, BN), 0)
          == ids[None, :]).astype(jnp.float32)
    ps = jnp.dot(oh, x_ref[...], preferred_element_type=jnp.float32)
    pc = jnp.sum(oh, axis=1)

    @pl.when(i == 0)
    def _():
        s_ref[...] = jnp.zeros_like(s_ref)
        c_ref[...] = jnp.zeros_like(c_ref)
    s_ref[...] += ps
    c_ref[...] += jnp.broadcast_to(pc[:, None], c_ref.shape)


def _pool_sums(x, batch):
    n, d = x.shape
    ids3 = batch.reshape(n // BN, 1, BN)
    return pl.pallas_call(
        _pool_body,
        grid=(n // BN,),
        in_specs=[
            pl.BlockSpec((1, 1, BN), lambda i: (i, 0, 0)),
            pl.BlockSpec((BN, d), lambda i: (i, 0)),
        ],
        out_specs=[
            pl.BlockSpec((NB, d), lambda i: (0, 0)),
            pl.BlockSpec((NB, 128), lambda i: (0, 0)),
        ],
        out_shape=[jax.ShapeDtypeStruct((NB, d), jnp.float32),
                   jax.ShapeDtypeStruct((NB, 128), jnp.float32)],
    )(ids3, x)


def _comb_body(ss_ref, sc_ref, ts_ref, tc_ref, o1_ref, o2_ref):
    x = (ss_ref[...] / jnp.maximum(sc_ref[:, 0:1], 1.0)
         + ts_ref[...] / jnp.maximum(tc_ref[:, 0:1], 1.0))
    o1_ref[...] = x
    o2_ref[...] = jax.nn.sigmoid(x)


def _combine(ss, sc, ts, tc):
    d = ss.shape[1]
    return pl.pallas_call(
        _comb_body,
        out_shape=[jax.ShapeDtypeStruct((NB, d), jnp.float32),
                   jax.ShapeDtypeStruct((NB, d), jnp.float32)],
    )(ss, sc, ts, tc)


def _pad_cols(w, tgt):
    return jnp.pad(w, ((0, 0), (0, tgt - w.shape[1])))


def _gat_layer(x, src, dst, ea, Wl, bl, Wr, br, We, att, bias):
    n = x.shape[0]
    e = src.shape[0]
    h, c = att.shape
    if h == 4:
        c_pad, F, nch = 256, 128, 8
    else:
        c_pad, F, nch = 1408, 128, 11
    hcp = h * c_pad

    Wlp = _pad_cols(Wl, hcp)
    Wrp = _pad_cols(Wr, hcp)
    W2 = jnp.concatenate([Wlp, Wrp], axis=1)
    b2 = jnp.concatenate([
        jnp.pad(bl, (0, hcp - bl.shape[0])),
        jnp.pad(br, (0, hcp - br.shape[0])),
    ])
    xlr = _matmul_bias(x, W2, b2)
    xlr_r = xlr.reshape(n * 2 * nch, F)

    ea_p = jnp.pad(ea, ((0, 0), (0, 16 - ea.shape[1])))
    We_p = _pad_cols(jnp.pad(We, ((0, 16 - We.shape[0]), (0, 0))), hcp)
    ee = _matmul_bias(ea_p, We_p, jnp.zeros((hcp,), jnp.float32), bm=2048)
    ee_r = ee.reshape(e * nch, F)

    att_flat = jnp.pad(att.reshape(-1), (0, hcp - h * c))
    bias_p = jnp.pad(bias, (0, hcp - bias.shape[0]))

    ex = _sc_attn(n, e, h, c_pad, F, nch)(xlr_r, ee_r, src, dst, att_flat)
    denP = _sc_den(n, e, h)(ex, dst)
    nchb = 2 * nch
    head_map = jnp.array([(f * 64) // c_pad for f in range(nchb)],
                         dtype=jnp.int32)
    exsel = ex[head_map].reshape(nchb, 1, e)
    den = (denP[0] + denP[1])[:, :h]
    densel = den.T[head_map].reshape(nchb, 1, n)
    y3 = _sc_aggr(n, e, h, c_pad, F, nch)(
        xlr_r, exsel, densel, src, dst, bias_p)
    return y3.transpose(1, 0, 2).reshape(n, hcp)


def _branch(x, ei, ea, params):
    src, dst = ei[0], ei[1]
    for (Wl, bl, Wr, br, We, att, bias) in params:
        x = _gat_layer(x, src, dst, ea, Wl, bl, Wr, br, We, att, bias)
    return x


def kernel(x_s, edge_index_s, edge_attr_s, x_t, edge_index_t, edge_attr_t, xs_batch, xt_batch, s1_Wl, s1_bl, s1_Wr, s1_br, s1_We, s1_att, s1_bias, s2_Wl, s2_bl, s2_Wr, s2_br, s2_We, s2_att, s2_bias, s3_Wl, s3_bl, s3_Wr, s3_br, s3_We, s3_att, s3_bias, t1_Wl, t1_bl, t1_Wr, t1_br, t1_We, t1_att, t1_bias, t2_Wl, t2_bl, t2_Wr, t2_br, t2_We, t2_att, t2_bias, t3_Wl, t3_bl, t3_Wr, t3_br, t3_We, t3_att, t3_bias):
    ps = [
        (s1_Wl, s1_bl, s1_Wr, s1_br, s1_We, s1_att, s1_bias),
        (s2_Wl, s2_bl, s2_Wr, s2_br, s2_We, s2_att, s2_bias),
        (s3_Wl, s3_bl, s3_Wr, s3_br, s3_We, s3_att, s3_bias),
    ]
    pt = [
        (t1_Wl, t1_bl, t1_Wr, t1_br, t1_We, t1_att, t1_bias),
        (t2_Wl, t2_bl, t2_Wr, t2_br, t2_We, t2_att, t2_bias),
        (t3_Wl, t3_bl, t3_Wr, t3_br, t3_We, t3_att, t3_bias),
    ]
    npad = 240
    x_s = jnp.pad(x_s, ((0, npad), (0, 0)))
    x_t = jnp.pad(x_t, ((0, npad), (0, 0)))
    xs_batch = jnp.pad(xs_batch, (0, npad), constant_values=NB)
    xt_batch = jnp.pad(xt_batch, (0, npad), constant_values=NB)
    xs = _branch(x_s, edge_index_s, edge_attr_s, ps)
    xt = _branch(x_t, edge_index_t, edge_attr_t, pt)
    ss, sc = _pool_sums(xs, xs_batch)
    ts, tc = _pool_sums(xt, xt_batch)
    x, sg = _combine(ss, sc, ts, tc)
    return (x[:, :OUT], sg[:, :OUT])
